# baseline (speedup 1.0000x reference)
import functools

import jax
import jax.numpy as jnp
from jax import lax
from jax.experimental import pallas as pl
from jax.experimental.pallas import tpu as pltpu
from jax.experimental.pallas import tpu_sc as plsc

_NC = 2
_NS = 16
_L = 16


def _segsum_sc(table, src_idx, dst, n, e, w, with_counts, split_edges):
    assert not (with_counts and split_edges)
    if split_edges:
        C = 40
        eper = e // (_NS * _NC)
    else:
        C = 80
        eper = e // _NS
    nchunks = eper // C
    rpt = -(-n // (128 * _NS)) * 128
    n_pad = rpt * _NS
    assert eper % C == 0 and w % 128 == 0

    mesh = plsc.VectorSubcoreMesh(core_axis_name="c", subcore_axis_name="s")
    B = 25
    nblocks = nchunks // B
    assert nchunks % B == 0

    out_type = [jax.ShapeDtypeStruct((_NC, n_pad, w), jnp.float32)]
    scratch = [
        pltpu.VMEM((B, C), jnp.int32),
        pltpu.VMEM((B, C), jnp.int32),
        pltpu.VMEM((B, C), jnp.int32),
        pltpu.VMEM((C, w), jnp.float32),
        pltpu.VMEM((C, w), jnp.float32),
        pltpu.VMEM((8, w), jnp.float32),
        pltpu.VMEM_SHARED((n_pad, w), jnp.float32),
        pltpu.SemaphoreType.DMA,
        pltpu.SemaphoreType.DMA,
        pltpu.SemaphoreType.DMA,
        pltpu.SemaphoreType.DMA,
    ]
    if with_counts:
        out_type.append(jax.ShapeDtypeStruct((_NS, 1, n_pad), jnp.float32))
        scratch.append(pltpu.VMEM((n_pad,), jnp.float32))

    def body(table_ref, src_ref, dst_ref, *rest):
        if with_counts:
            (sums_out, cnt_out, src_blk, dst_blkA, dst_blkB, rows0_v,
             rows1_v, zero_v, accum_sh, gsem0, gsem1, ssem0, ssem1,
             cnt_local) = rest
        else:
            (sums_out, src_blk, dst_blkA, dst_blkB, rows0_v, rows1_v,
             zero_v, accum_sh, gsem0, gsem1, ssem0, ssem1) = rest
        cid = lax.axis_index("c")
        sid = lax.axis_index("s")

        row0 = sid * rpt

        ones16 = jnp.full((_L,), 1.0, jnp.float32)

        def load_block(blk, dstb):
            pltpu.sync_copy(src_ref.at[cid, sid, blk], src_blk)
            if split_edges:
                pltpu.sync_copy(dst_ref.at[cid, sid, blk], dstb)
            else:
                pltpu.sync_copy(dst_ref.at[sid, blk], dstb)
            if with_counts:
                @pl.when(cid == 0)
                def _():
                    for j in range(B):
                        for k in range(C // _L):
                            idx = dstb[j, pl.ds(k * _L, _L)]
                            plsc.addupdate_scatter(cnt_local, [idx], ones16)

        for i in range(8):
            for j in range(w // _L):
                zero_v[i, pl.ds(j * _L, _L)] = jnp.zeros((_L,), jnp.float32)

        def zacc(r, _):
            pltpu.sync_copy(zero_v, accum_sh.at[pl.ds(row0 + r * 8, 8)])
            return 0
        lax.fori_loop(0, rpt // 8, zacc, 0)

        if with_counts:
            def zcnt(i, _):
                cnt_local[pl.ds(i * _L, _L)] = jnp.zeros((_L,), jnp.float32)
                return 0
            lax.fori_loop(0, n_pad // _L, zcnt, 0)

        plsc.subcore_barrier()

        load_block(0, dst_blkA)
        pltpu.async_copy(table_ref.at[src_blk.at[0]], rows0_v, gsem0)

        def make_phase(rows_v, gsem, ssem, orows_v, ogsem, ossem):
            def phase(i):
                pltpu.make_async_copy(table_ref.at[pl.ds(0, C)], rows_v,
                                      gsem).wait()

                @pl.when(i > 0)
                def _():
                    pltpu.make_async_copy(orows_v,
                                          accum_sh.at[dst_blkA.at[0]],
                                          ossem).wait()

                nxt = i + 1
                crossing = jnp.logical_and(nxt % B == 0, nxt < nchunks)
                pb_a = (i // B) % 2 == 0

                @pl.when(jnp.logical_and(crossing, pb_a))
                def _():
                    load_block(nxt // B, dst_blkB)

                @pl.when(jnp.logical_and(crossing, jnp.logical_not(pb_a)))
                def _():
                    load_block(nxt // B, dst_blkA)

                @pl.when(nxt < nchunks)
                def _():
                    pltpu.async_copy(table_ref.at[src_blk.at[nxt % B]],
                                     orows_v, ogsem)

                @pl.when(pb_a)
                def _():
                    pltpu.async_copy(rows_v, accum_sh.at[dst_blkA.at[i % B]],
                                     ssem, add=True)

                @pl.when(jnp.logical_not(pb_a))
                def _():
                    pltpu.async_copy(rows_v, accum_sh.at[dst_blkB.at[i % B]],
                                     ssem, add=True)
            return phase

        phase0 = make_phase(rows0_v, gsem0, ssem0, rows1_v, gsem1, ssem1)
        phase1 = make_phase(rows1_v, gsem1, ssem1, rows0_v, gsem0, ssem0)

        def step(i, _):
            @pl.when(i % 2 == 0)
            def _():
                phase0(i)

            @pl.when(i % 2 == 1)
            def _():
                phase1(i)
            return 0
        lax.fori_loop(0, nchunks, step, 0)

        last_ssem = ssem0 if (nchunks - 1) % 2 == 0 else ssem1
        last_rows = rows0_v if (nchunks - 1) % 2 == 0 else rows1_v
        pltpu.make_async_copy(last_rows, accum_sh.at[dst_blkA.at[0]],
                              last_ssem).wait()

        plsc.subcore_barrier()

        pltpu.sync_copy(accum_sh.at[pl.ds(row0, rpt)],
                        sums_out.at[cid, pl.ds(row0, rpt)])

        if with_counts:
            @pl.when(cid == 0)
            def _():
                pltpu.sync_copy(cnt_local, cnt_out.at[sid, 0])

    fn = pl.kernel(body, out_type=tuple(out_type), mesh=mesh,
                   scratch_types=scratch,
                   compiler_params=pltpu.CompilerParams(
                       needs_layout_passes=False))
    src5 = src_idx.reshape(_NC, _NS, nblocks, B, C)
    if split_edges:
        dstr = dst.reshape(_NC, _NS, nblocks, B, C)
    else:
        dstr = dst.reshape(_NS, nblocks, B, C)
    return fn(table, src5, dstr)


def _tc_layer(x, s, cnt, W1l, b1, W1r, W2l, b2, W2r):
    n, din = x.shape
    dh = W1l.shape[1]
    do = W2l.shape[1]
    hw = din // 2
    G = 1000
    grid = (n // G,)

    def body(x_ref, s_ref, cnt_ref, w1l_ref, b1_ref, w1r_ref,
             w2l_ref, b2_ref, w2r_ref, p_ref, q_ref):
        c = jnp.sum(cnt_ref[...], axis=1, keepdims=True)
        r = 1.0 / jnp.maximum(c, 1.0)
        a0 = s_ref[0] * r
        a1 = s_ref[1] * r
        h = (jnp.dot(a0, w1l_ref[0:hw, :], preferred_element_type=jnp.float32)
             + jnp.dot(a1, w1l_ref[hw:din, :],
                       preferred_element_type=jnp.float32)
             + jnp.dot(x_ref[...], w1r_ref[...],
                       preferred_element_type=jnp.float32)
             + b1_ref[...])
        h = jnp.maximum(h, 0.0)
        p_ref[...] = jnp.dot(h, w2l_ref[...], preferred_element_type=jnp.float32)
        q_ref[...] = (jnp.dot(h, w2r_ref[...],
                              preferred_element_type=jnp.float32)
                      + b2_ref[...])

    zero2 = lambda i: (0, 0)
    return pl.pallas_call(
        body,
        grid=grid,
        in_specs=[
            pl.BlockSpec((G, din), lambda i: (i, 0)),
            pl.BlockSpec((2, G, hw), lambda i: (0, i, 0)),
            pl.BlockSpec((G, _L), lambda i: (i, 0)),
            pl.BlockSpec((din, dh), zero2),
            pl.BlockSpec((1, dh), zero2),
            pl.BlockSpec((din, dh), zero2),
            pl.BlockSpec((dh, do), zero2),
            pl.BlockSpec((1, do), zero2),
            pl.BlockSpec((dh, do), zero2),
        ],
        out_specs=[
            pl.BlockSpec((G, do), lambda i: (i, 0)),
            pl.BlockSpec((G, do), lambda i: (i, 0)),
        ],
        out_shape=[
            jax.ShapeDtypeStruct((n, do), jnp.float32),
            jax.ShapeDtypeStruct((n, do), jnp.float32),
        ],
    )(x, s, cnt, W1l, b1.reshape(1, dh), W1r, W2l, b2.reshape(1, do), W2r)


def _tc_epilogue(s2, cnt, q):
    n, do = q.shape
    G = 1000
    grid = (n // G,)

    def body(s2_ref, cnt_ref, q_ref, out_ref):
        c = jnp.sum(cnt_ref[...], axis=1, keepdims=True)
        r = 1.0 / jnp.maximum(c, 1.0)
        out_ref[...] = (s2_ref[0] + s2_ref[1]) * r + q_ref[...]

    return pl.pallas_call(
        body,
        grid=grid,
        in_specs=[
            pl.BlockSpec((2, G, do), lambda i: (0, i, 0)),
            pl.BlockSpec((G, _L), lambda i: (i, 0)),
            pl.BlockSpec((G, do), lambda i: (i, 0)),
        ],
        out_specs=pl.BlockSpec((G, do), lambda i: (i, 0)),
        out_shape=jax.ShapeDtypeStruct((n, do), jnp.float32),
    )(s2, cnt, q)


def kernel(x, edge_index, W1l, b1, W1r, W2l, b2, W2r):
    n, din = x.shape
    e = edge_index.shape[1]
    src = edge_index[0]
    dst = edge_index[1]
    src2 = jnp.concatenate([src, src + n])

    hw = din // 2
    xs = jnp.concatenate([x[:, :hw], x[:, hw:]], axis=0)
    sums1, cntp = _segsum_sc(xs, src2, dst, n, e, hw, True, False)
    cnt = cntp.reshape(_NS, -1).T[:n]

    p, q = _tc_layer(x, sums1, cnt, W1l, b1, W1r, W2l, b2, W2r)

    (sums2,) = _segsum_sc(p, src, dst, n, e, p.shape[1], False, True)

    return _tc_epilogue(sums2, cnt, q)


# speedup vs baseline: 1.1761x; 1.1761x over previous
import functools

import jax
import jax.numpy as jnp
from jax import lax
from jax.experimental import pallas as pl
from jax.experimental.pallas import tpu as pltpu
from jax.experimental.pallas import tpu_sc as plsc

_NC = 2
_NS = 16
_L = 16


def _segsum_sc(table, src_idx, dst, n, e, w, with_counts, split_edges):
    assert not (with_counts and split_edges)
    if split_edges:
        C = 125
        eper = e // (_NS * _NC)
    else:
        C = 80
        eper = e // _NS
    nchunks = eper // C
    rpt = -(-n // (128 * _NS)) * 128
    n_pad = rpt * _NS
    assert eper % C == 0 and w % 128 == 0

    mesh = plsc.VectorSubcoreMesh(core_axis_name="c", subcore_axis_name="s")
    B = 25 if not split_edges else 20
    nblocks = nchunks // B
    assert nchunks % B == 0

    out_type = [jax.ShapeDtypeStruct((_NC, n_pad, w), jnp.float32)]
    scratch = [
        pltpu.VMEM((B, C), jnp.int32),
        pltpu.VMEM((B, C), jnp.int32),
        pltpu.VMEM((B, C), jnp.int32),
        pltpu.VMEM((C, w), jnp.float32),
        pltpu.VMEM((C, w), jnp.float32),
        pltpu.VMEM((8, w), jnp.float32),
        pltpu.VMEM_SHARED((n_pad, w), jnp.float32),
        pltpu.SemaphoreType.DMA,
        pltpu.SemaphoreType.DMA,
        pltpu.SemaphoreType.DMA,
        pltpu.SemaphoreType.DMA,
    ]
    if with_counts:
        out_type.append(jax.ShapeDtypeStruct((_NS, 1, n_pad), jnp.float32))
        scratch.append(pltpu.VMEM((n_pad,), jnp.float32))

    def body(table_ref, src_ref, dst_ref, *rest):
        if with_counts:
            (sums_out, cnt_out, src_blk, dst_blkA, dst_blkB, rows0_v,
             rows1_v, zero_v, accum_sh, gsem0, gsem1, ssem0, ssem1,
             cnt_local) = rest
        else:
            (sums_out, src_blk, dst_blkA, dst_blkB, rows0_v, rows1_v,
             zero_v, accum_sh, gsem0, gsem1, ssem0, ssem1) = rest
        cid = lax.axis_index("c")
        sid = lax.axis_index("s")

        row0 = sid * rpt

        ones16 = jnp.full((_L,), 1.0, jnp.float32)

        def load_block(blk, dstb):
            pltpu.sync_copy(src_ref.at[cid, sid, blk], src_blk)
            if split_edges:
                pltpu.sync_copy(dst_ref.at[cid, sid, blk], dstb)
            else:
                pltpu.sync_copy(dst_ref.at[sid, blk], dstb)
            if with_counts:
                @pl.when(cid == 0)
                def _():
                    for j in range(B):
                        for k in range(C // _L):
                            idx = dstb[j, pl.ds(k * _L, _L)]
                            plsc.addupdate_scatter(cnt_local, [idx], ones16)

        for i in range(8):
            for j in range(w // _L):
                zero_v[i, pl.ds(j * _L, _L)] = jnp.zeros((_L,), jnp.float32)

        def zacc(r, _):
            pltpu.sync_copy(zero_v, accum_sh.at[pl.ds(row0 + r * 8, 8)])
            return 0
        lax.fori_loop(0, rpt // 8, zacc, 0)

        if with_counts:
            def zcnt(i, _):
                cnt_local[pl.ds(i * _L, _L)] = jnp.zeros((_L,), jnp.float32)
                return 0
            lax.fori_loop(0, n_pad // _L, zcnt, 0)

        plsc.subcore_barrier()

        load_block(0, dst_blkA)
        pltpu.async_copy(table_ref.at[src_blk.at[0]], rows0_v, gsem0)

        def make_phase(rows_v, gsem, ssem, orows_v, ogsem, ossem):
            def phase(i):
                pltpu.make_async_copy(table_ref.at[src_blk.at[0]], rows_v,
                                      gsem).wait()

                @pl.when(i > 0)
                def _():
                    pltpu.make_async_copy(orows_v,
                                          accum_sh.at[dst_blkA.at[0]],
                                          ossem).wait()

                nxt = i + 1
                crossing = jnp.logical_and(nxt % B == 0, nxt < nchunks)
                pb_a = (i // B) % 2 == 0

                @pl.when(jnp.logical_and(crossing, pb_a))
                def _():
                    load_block(nxt // B, dst_blkB)

                @pl.when(jnp.logical_and(crossing, jnp.logical_not(pb_a)))
                def _():
                    load_block(nxt // B, dst_blkA)

                @pl.when(nxt < nchunks)
                def _():
                    pltpu.async_copy(table_ref.at[src_blk.at[nxt % B]],
                                     orows_v, ogsem)

                @pl.when(pb_a)
                def _():
                    pltpu.async_copy(rows_v, accum_sh.at[dst_blkA.at[i % B]],
                                     ssem, add=True)

                @pl.when(jnp.logical_not(pb_a))
                def _():
                    pltpu.async_copy(rows_v, accum_sh.at[dst_blkB.at[i % B]],
                                     ssem, add=True)
            return phase

        phase0 = make_phase(rows0_v, gsem0, ssem0, rows1_v, gsem1, ssem1)
        phase1 = make_phase(rows1_v, gsem1, ssem1, rows0_v, gsem0, ssem0)

        def step(i, _):
            @pl.when(i % 2 == 0)
            def _():
                phase0(i)

            @pl.when(i % 2 == 1)
            def _():
                phase1(i)
            return 0
        lax.fori_loop(0, nchunks, step, 0)

        last_ssem = ssem0 if (nchunks - 1) % 2 == 0 else ssem1
        last_rows = rows0_v if (nchunks - 1) % 2 == 0 else rows1_v
        pltpu.make_async_copy(last_rows, accum_sh.at[dst_blkA.at[0]],
                              last_ssem).wait()

        plsc.subcore_barrier()

        pltpu.sync_copy(accum_sh.at[pl.ds(row0, rpt)],
                        sums_out.at[cid, pl.ds(row0, rpt)])

        if with_counts:
            @pl.when(cid == 0)
            def _():
                pltpu.sync_copy(cnt_local, cnt_out.at[sid, 0])

    fn = pl.kernel(body, out_type=tuple(out_type), mesh=mesh,
                   scratch_types=scratch,
                   compiler_params=pltpu.CompilerParams(
                       needs_layout_passes=False))
    src5 = src_idx.reshape(_NC, _NS, nblocks, B, C)
    if split_edges:
        dstr = dst.reshape(_NC, _NS, nblocks, B, C)
    else:
        dstr = dst.reshape(_NS, nblocks, B, C)
    return fn(table, src5, dstr)


def _tc_layer(x, s, cnt, W1l, b1, W1r, W2l, b2, W2r):
    n, din = x.shape
    dh = W1l.shape[1]
    do = W2l.shape[1]
    hw = din // 2
    G = 1000
    grid = (n // G,)

    def body(x_ref, s_ref, cnt_ref, w1l_ref, b1_ref, w1r_ref,
             w2l_ref, b2_ref, w2r_ref, p_ref, q_ref):
        c = jnp.sum(cnt_ref[...], axis=1, keepdims=True)
        r = 1.0 / jnp.maximum(c, 1.0)
        a0 = s_ref[0] * r
        a1 = s_ref[1] * r
        h = (jnp.dot(a0, w1l_ref[0:hw, :], preferred_element_type=jnp.float32)
             + jnp.dot(a1, w1l_ref[hw:din, :],
                       preferred_element_type=jnp.float32)
             + jnp.dot(x_ref[...], w1r_ref[...],
                       preferred_element_type=jnp.float32)
             + b1_ref[...])
        h = jnp.maximum(h, 0.0)
        p_ref[...] = jnp.dot(h, w2l_ref[...], preferred_element_type=jnp.float32)
        q_ref[...] = (jnp.dot(h, w2r_ref[...],
                              preferred_element_type=jnp.float32)
                      + b2_ref[...])

    zero2 = lambda i: (0, 0)
    return pl.pallas_call(
        body,
        grid=grid,
        in_specs=[
            pl.BlockSpec((G, din), lambda i: (i, 0)),
            pl.BlockSpec((2, G, hw), lambda i: (0, i, 0)),
            pl.BlockSpec((G, _L), lambda i: (i, 0)),
            pl.BlockSpec((din, dh), zero2),
            pl.BlockSpec((1, dh), zero2),
            pl.BlockSpec((din, dh), zero2),
            pl.BlockSpec((dh, do), zero2),
            pl.BlockSpec((1, do), zero2),
            pl.BlockSpec((dh, do), zero2),
        ],
        out_specs=[
            pl.BlockSpec((G, do), lambda i: (i, 0)),
            pl.BlockSpec((G, do), lambda i: (i, 0)),
        ],
        out_shape=[
            jax.ShapeDtypeStruct((n, do), jnp.float32),
            jax.ShapeDtypeStruct((n, do), jnp.float32),
        ],
    )(x, s, cnt, W1l, b1.reshape(1, dh), W1r, W2l, b2.reshape(1, do), W2r)


def _tc_epilogue(s2, cnt, q):
    n, do = q.shape
    G = 1000
    grid = (n // G,)

    def body(s2_ref, cnt_ref, q_ref, out_ref):
        c = jnp.sum(cnt_ref[...], axis=1, keepdims=True)
        r = 1.0 / jnp.maximum(c, 1.0)
        out_ref[...] = (s2_ref[0] + s2_ref[1]) * r + q_ref[...]

    return pl.pallas_call(
        body,
        grid=grid,
        in_specs=[
            pl.BlockSpec((2, G, do), lambda i: (0, i, 0)),
            pl.BlockSpec((G, _L), lambda i: (i, 0)),
            pl.BlockSpec((G, do), lambda i: (i, 0)),
        ],
        out_specs=pl.BlockSpec((G, do), lambda i: (i, 0)),
        out_shape=jax.ShapeDtypeStruct((n, do), jnp.float32),
    )(s2, cnt, q)


def kernel(x, edge_index, W1l, b1, W1r, W2l, b2, W2r):
    n, din = x.shape
    e = edge_index.shape[1]
    src = edge_index[0]
    dst = edge_index[1]
    src2 = jnp.concatenate([src, src + n])

    hw = din // 2
    xs = jnp.concatenate([x[:, :hw], x[:, hw:]], axis=0)
    sums1, cntp = _segsum_sc(xs, src2, dst, n, e, hw, True, False)
    cnt = cntp.reshape(_NS, -1).T[:n]

    p, q = _tc_layer(x, sums1, cnt, W1l, b1, W1r, W2l, b2, W2r)

    (sums2,) = _segsum_sc(p, src, dst, n, e, p.shape[1], False, True)

    return _tc_epilogue(sums2, cnt, q)
